# Initial kernel scaffold; baseline (speedup 1.0000x reference)
import jax
import jax.numpy as jnp
from jax.experimental import pallas as pl


def kernel(X, table):
    raise NotImplementedError("write your pallas kernel here")


# speedup vs baseline: 4.2697x; 4.2697x over previous
import functools

import jax
import jax.numpy as jnp
from jax import lax
from jax.experimental import pallas as pl
from jax.experimental.pallas import tpu as pltpu
from jax.experimental.pallas import tpu_sc as plsc

_NUM_WORKERS = 32
_CHUNK = 128


def _sc_gather(idx3, table):
    n_workers, n_chunks, chunk = idx3.shape
    _, dp = table.shape
    mesh = plsc.VectorSubcoreMesh(core_axis_name="c", subcore_axis_name="s")

    @functools.partial(
        pl.kernel,
        mesh=mesh,
        out_type=jax.ShapeDtypeStruct((n_workers, n_chunks, chunk, dp),
                                      jnp.float32),
        scratch_types=[
            pltpu.VMEM((n_chunks, chunk), jnp.int32),
            pltpu.VMEM((chunk, dp), jnp.float32),
            pltpu.VMEM((chunk, dp), jnp.float32),
            pltpu.SemaphoreType.DMA,
            pltpu.SemaphoreType.DMA,
        ],
    )
    def k(x_hbm, tbl_hbm, out_hbm, idx_v, buf_a, buf_b, sem_a, sem_b):
        wid = lax.axis_index("s") * 2 + lax.axis_index("c")
        pltpu.sync_copy(x_hbm.at[wid], idx_v)
        pltpu.async_copy(tbl_hbm.at[idx_v.at[0]], buf_a, sem_a)

        def body(i, carry):
            g = 2 * i
            pltpu.async_copy(tbl_hbm.at[idx_v.at[g + 1]], buf_b, sem_b)
            pltpu.make_async_copy(tbl_hbm.at[idx_v.at[g]], buf_a,
                                  sem_a).wait()
            pltpu.sync_copy(buf_a, out_hbm.at[wid, g])

            @pl.when(g + 2 < n_chunks)
            def _():
                pltpu.async_copy(tbl_hbm.at[idx_v.at[g + 2]], buf_a, sem_a)

            pltpu.make_async_copy(tbl_hbm.at[idx_v.at[g + 1]], buf_b,
                                  sem_b).wait()
            pltpu.sync_copy(buf_b, out_hbm.at[wid, g + 1])
            return carry

        lax.fori_loop(0, n_chunks // 2, body, 0)

    return k(idx3, table)


def kernel(X, table):
    b, t = X.shape
    _, d = table.shape
    total = b * t
    n_chunks = total // (_NUM_WORKERS * _CHUNK)
    idx3 = X.reshape(_NUM_WORKERS, n_chunks, _CHUNK).astype(jnp.int32)
    table_p = jnp.pad(table, ((0, 0), (0, 128 - d)))
    out = _sc_gather(idx3, table_p)
    return out.reshape(b * t, 128)[:, :d].reshape(b, t, d)


# speedup vs baseline: 4.9955x; 1.1700x over previous
import functools

import jax
import jax.numpy as jnp
from jax import lax
from jax.experimental import pallas as pl
from jax.experimental.pallas import tpu as pltpu
from jax.experimental.pallas import tpu_sc as plsc

_NUM_WORKERS = 32
_CHUNK = 128


def _pad_body(t_ref, o_ref):
    o_ref[:, : t_ref.shape[1]] = t_ref[...]


def _tc_pad(table, dp):
    v, d = table.shape
    blk = 10000
    return pl.pallas_call(
        _pad_body,
        grid=(v // blk,),
        in_specs=[pl.BlockSpec((blk, d), lambda i: (i, 0))],
        out_specs=pl.BlockSpec((blk, dp), lambda i: (i, 0)),
        out_shape=jax.ShapeDtypeStruct((v, dp), jnp.float32),
    )(table)


def _sc_gather(idx3, table):
    n_workers, n_chunks, chunk = idx3.shape
    _, dp = table.shape
    mesh = plsc.VectorSubcoreMesh(core_axis_name="c", subcore_axis_name="s")

    @functools.partial(
        pl.kernel,
        mesh=mesh,
        out_type=jax.ShapeDtypeStruct((n_workers * n_chunks * chunk, dp),
                                      jnp.float32),
        scratch_types=[
            pltpu.VMEM((n_chunks, chunk), jnp.int32),
            pltpu.VMEM((chunk, dp), jnp.float32),
            pltpu.VMEM((chunk, dp), jnp.float32),
            pltpu.SemaphoreType.DMA,
            pltpu.SemaphoreType.DMA,
        ],
    )
    def k(x_hbm, tbl_hbm, out_hbm, idx_v, buf_a, buf_b, sem_a, sem_b):
        wid = lax.axis_index("s") * 2 + lax.axis_index("c")
        base = wid * (n_chunks * chunk)
        pltpu.sync_copy(x_hbm.at[wid], idx_v)
        pltpu.async_copy(tbl_hbm.at[idx_v.at[0]], buf_a, sem_a)

        def body(i, carry):
            g = 2 * i
            pltpu.async_copy(tbl_hbm.at[idx_v.at[g + 1]], buf_b, sem_b)
            pltpu.make_async_copy(tbl_hbm.at[idx_v.at[g]], buf_a,
                                  sem_a).wait()
            pltpu.sync_copy(buf_a, out_hbm.at[pl.ds(base + g * chunk, chunk)])

            @pl.when(g + 2 < n_chunks)
            def _():
                pltpu.async_copy(tbl_hbm.at[idx_v.at[g + 2]], buf_a, sem_a)

            pltpu.make_async_copy(tbl_hbm.at[idx_v.at[g + 1]], buf_b,
                                  sem_b).wait()
            pltpu.sync_copy(buf_b,
                            out_hbm.at[pl.ds(base + (g + 1) * chunk, chunk)])
            return carry

        lax.fori_loop(0, n_chunks // 2, body, 0)

    return k(idx3, table)


def kernel(X, table):
    b, t = X.shape
    _, d = table.shape
    total = b * t
    n_chunks = total // (_NUM_WORKERS * _CHUNK)
    idx3 = X.reshape(_NUM_WORKERS, n_chunks, _CHUNK).astype(jnp.int32)
    table_p = _tc_pad(table, 128)
    out = _sc_gather(idx3, table_p)
    return out[:, :d].reshape(b, t, d)
